# baseline (speedup 1.0000x reference)
import jax
import jax.numpy as jnp
from jax import lax
from jax.experimental import pallas as pl
from jax.experimental.pallas import tpu as pltpu
from jax.experimental.pallas import tpu_sc as plsc

VOCAB = 100000
EMBED_DIM = 128
BATCH = 4096
N_FIELDS = 26

NUM_CORES = 1
NUM_SUBCORES = 16
NUM_WORKERS = NUM_CORES * NUM_SUBCORES
TOTAL_ROWS = BATCH * N_FIELDS
ROWS_PER_WORKER = TOTAL_ROWS // NUM_WORKERS
BATCHES_PER_WORKER = BATCH // NUM_WORKERS
BATCHES_PER_CHUNK = 4
CHUNK = BATCHES_PER_CHUNK * N_FIELDS
CHUNKS_PER_WORKER = BATCHES_PER_WORKER // BATCHES_PER_CHUNK
NBUF = 4


def _body(idx_hbm, table_hbm, out_hbm, idx_v, rows_v, gsem, osem):
    c = lax.axis_index("c")
    s = lax.axis_index("s")
    wid = s * NUM_CORES + c
    pltpu.sync_copy(idx_hbm.at[pl.ds(wid * ROWS_PER_WORKER, ROWS_PER_WORKER)], idx_v)
    bbase = wid * BATCHES_PER_WORKER

    def fire_gather(chunk, b):
        pltpu.async_copy(
            table_hbm.at[idx_v.at[pl.ds(chunk * CHUNK, CHUNK)]],
            rows_v.at[b],
            gsem.at[b],
        )

    def process(chunk, b, fire_next):
        pltpu.make_async_copy(
            table_hbm.at[idx_v.at[pl.ds(chunk * CHUNK, CHUNK)]],
            rows_v.at[b],
            gsem.at[b],
        ).wait()
        cps = []
        for k in range(BATCHES_PER_CHUNK):
            cp = pltpu.make_async_copy(
                rows_v.at[b, pl.ds(k * N_FIELDS, N_FIELDS)],
                out_hbm.at[bbase + chunk * BATCHES_PER_CHUNK + k],
                osem.at[b],
            )
            cp.start()
            cps.append(cp)
        for cp in cps:
            cp.wait()
        if fire_next:
            fire_gather(chunk + NBUF, b)

    for b in range(NBUF):
        fire_gather(b, b)

    def loop_body(g, carry):
        for b in range(NBUF):
            process(g * NBUF + b, b, True)
        return carry

    n_main_groups = CHUNKS_PER_WORKER // NBUF - 1
    lax.fori_loop(0, n_main_groups, loop_body, 0)
    for b in range(NBUF):
        process((CHUNKS_PER_WORKER - NBUF) + b, b, False)


@jax.jit
def _gather(idx, table):
    mesh = plsc.VectorSubcoreMesh(
        core_axis_name="c", subcore_axis_name="s", num_cores=NUM_CORES
    )
    return pl.kernel(
        _body,
        out_type=jax.ShapeDtypeStruct((BATCH, N_FIELDS, EMBED_DIM), jnp.float32),
        mesh=mesh,
        compiler_params=pltpu.CompilerParams(use_tc_tiling_on_sc=True),
        scratch_types=[
            pltpu.VMEM((ROWS_PER_WORKER,), jnp.int32),
            pltpu.VMEM((NBUF, CHUNK, EMBED_DIM), jnp.float32),
            pltpu.SemaphoreType.DMA((NBUF,)),
            pltpu.SemaphoreType.DMA((NBUF,)),
        ],
    )(idx, table)


def kernel(indices, table):
    idx = indices.astype(jnp.int32).reshape(TOTAL_ROWS)
    return _gather(idx, table)


# speedup vs baseline: 1.9442x; 1.9442x over previous
import jax
import jax.numpy as jnp
from jax import lax
from jax.experimental import pallas as pl
from jax.experimental.pallas import tpu as pltpu
from jax.experimental.pallas import tpu_sc as plsc

VOCAB = 100000
EMBED_DIM = 128
BATCH = 4096
N_FIELDS = 26

NUM_CORES = 2
NUM_SUBCORES = 16
NUM_WORKERS = NUM_CORES * NUM_SUBCORES
TOTAL_ROWS = BATCH * N_FIELDS
ROWS_PER_WORKER = TOTAL_ROWS // NUM_WORKERS
CHUNK = 104
CHUNKS_PER_WORKER = ROWS_PER_WORKER // CHUNK
NBUF = 4


def _body(idx_hbm, table_hbm, out_hbm, idx_v, rows_v, gsem, osem):
    c = lax.axis_index("c")
    s = lax.axis_index("s")
    wid = s * NUM_CORES + c
    pltpu.sync_copy(idx_hbm.at[pl.ds(wid * ROWS_PER_WORKER, ROWS_PER_WORKER)], idx_v)
    base = wid * ROWS_PER_WORKER

    def fire_gather(chunk, b):
        pltpu.async_copy(
            table_hbm.at[idx_v.at[pl.ds(chunk * CHUNK, CHUNK)]],
            rows_v.at[b],
            gsem.at[b],
        )

    def process(chunk, b, fire_next):
        pltpu.make_async_copy(
            table_hbm.at[idx_v.at[pl.ds(chunk * CHUNK, CHUNK)]],
            rows_v.at[b],
            gsem.at[b],
        ).wait()
        cp = pltpu.make_async_copy(
            rows_v.at[b],
            out_hbm.at[pl.ds(base + chunk * CHUNK, CHUNK)],
            osem.at[b],
        )
        cp.start()
        cp.wait()
        if fire_next:
            fire_gather(chunk + NBUF, b)

    for b in range(NBUF):
        fire_gather(b, b)

    def loop_body(g, carry):
        for b in range(NBUF):
            process(g * NBUF + b, b, True)
        return carry

    n_main_groups = CHUNKS_PER_WORKER // NBUF - 1
    lax.fori_loop(0, n_main_groups, loop_body, 0)
    for b in range(NBUF):
        process((CHUNKS_PER_WORKER - NBUF) + b, b, False)


@jax.jit
def _gather(idx, table):
    mesh = plsc.VectorSubcoreMesh(
        core_axis_name="c", subcore_axis_name="s", num_cores=NUM_CORES
    )
    return pl.kernel(
        _body,
        out_type=jax.ShapeDtypeStruct((TOTAL_ROWS, EMBED_DIM), jnp.float32),
        mesh=mesh,
        compiler_params=pltpu.CompilerParams(use_tc_tiling_on_sc=True),
        scratch_types=[
            pltpu.VMEM((ROWS_PER_WORKER,), jnp.int32),
            pltpu.VMEM((NBUF, CHUNK, EMBED_DIM), jnp.float32),
            pltpu.SemaphoreType.DMA((NBUF,)),
            pltpu.SemaphoreType.DMA((NBUF,)),
        ],
    )(idx, table)


def kernel(indices, table):
    idx_t = indices.astype(jnp.int32).T.reshape(TOTAL_ROWS)
    out = _gather(idx_t, table)
    return out.reshape(N_FIELDS, BATCH, EMBED_DIM).transpose(1, 0, 2)


# speedup vs baseline: 1.9563x; 1.0063x over previous
import jax
import jax.numpy as jnp
from jax import lax
from jax.experimental import pallas as pl
from jax.experimental.pallas import tpu as pltpu
from jax.experimental.pallas import tpu_sc as plsc

VOCAB = 100000
EMBED_DIM = 128
BATCH = 4096
N_FIELDS = 26

NUM_CORES = 2
NUM_SUBCORES = 16
NUM_WORKERS = NUM_CORES * NUM_SUBCORES
TOTAL_ROWS = BATCH * N_FIELDS
ROWS_PER_WORKER = TOTAL_ROWS // NUM_WORKERS
CHUNK = 104
CHUNKS_PER_WORKER = ROWS_PER_WORKER // CHUNK
NBUF = 8


def _body(idx_hbm, table_hbm, out_hbm, idx_v, rows_v, gsem, osem):
    c = lax.axis_index("c")
    s = lax.axis_index("s")
    wid = s * NUM_CORES + c
    pltpu.sync_copy(idx_hbm.at[pl.ds(wid * ROWS_PER_WORKER, ROWS_PER_WORKER)], idx_v)
    base = wid * ROWS_PER_WORKER

    def fire_gather(chunk, b):
        pltpu.async_copy(
            table_hbm.at[idx_v.at[pl.ds(chunk * CHUNK, CHUNK)]],
            rows_v.at[b],
            gsem.at[b],
        )

    def process(chunk, b, fire_next):
        pltpu.make_async_copy(
            table_hbm.at[idx_v.at[pl.ds(chunk * CHUNK, CHUNK)]],
            rows_v.at[b],
            gsem.at[b],
        ).wait()
        cp = pltpu.make_async_copy(
            rows_v.at[b],
            out_hbm.at[pl.ds(base + chunk * CHUNK, CHUNK)],
            osem.at[b],
        )
        cp.start()
        cp.wait()
        if fire_next:
            fire_gather(chunk + NBUF, b)

    for b in range(NBUF):
        fire_gather(b, b)

    def loop_body(g, carry):
        for b in range(NBUF):
            process(g * NBUF + b, b, True)
        return carry

    n_main_groups = CHUNKS_PER_WORKER // NBUF - 1
    lax.fori_loop(0, n_main_groups, loop_body, 0)
    for b in range(NBUF):
        process((CHUNKS_PER_WORKER - NBUF) + b, b, False)


@jax.jit
def _gather(idx, table):
    mesh = plsc.VectorSubcoreMesh(
        core_axis_name="c", subcore_axis_name="s", num_cores=NUM_CORES
    )
    return pl.kernel(
        _body,
        out_type=jax.ShapeDtypeStruct((TOTAL_ROWS, EMBED_DIM), jnp.float32),
        mesh=mesh,
        compiler_params=pltpu.CompilerParams(use_tc_tiling_on_sc=True),
        scratch_types=[
            pltpu.VMEM((ROWS_PER_WORKER,), jnp.int32),
            pltpu.VMEM((NBUF, CHUNK, EMBED_DIM), jnp.float32),
            pltpu.SemaphoreType.DMA((NBUF,)),
            pltpu.SemaphoreType.DMA((NBUF,)),
        ],
    )(idx, table)


def kernel(indices, table):
    idx_t = indices.astype(jnp.int32).T.reshape(TOTAL_ROWS)
    out = _gather(idx_t, table)
    return out.reshape(N_FIELDS, BATCH, EMBED_DIM).transpose(1, 0, 2)
